# baseline (speedup 1.0000x reference)
import functools

import numpy as np
import jax
import jax.numpy as jnp
from jax import lax
from jax.experimental import pallas as pl
from jax.experimental.pallas import tpu as pltpu
from jax.experimental.pallas import tpu_sc as plsc

NUM_LEVELS = 16
T_ROWS = 2 ** 19
ROW_MASK = T_ROWS - 1
N_PTS = 1048576
OUT_COLS = 12 + 2 * NUM_LEVELS

P1 = np.int32(np.uint32(2654435761))
P2 = np.int32(805459861)
PI = 3.14159265358979

NUM_CORES = 2
NUM_SUBCORES = 16
NW = NUM_CORES * NUM_SUBCORES
PW = N_PTS // NW
LANES = 16
CHUNK = 128
NGRP = CHUNK // LANES
NCHUNK = PW // CHUNK
CB = 8 * CHUNK

_DENSE_LEVELS = 3

SH_ROWS = (5224, 37064, 278920, T_ROWS)
SH_BASE = (0, 5224, 42288, 321208)
SH_TOTAL = 845496
N_CACHED = 4


def _sin2pi(u):
    offs = jnp.where(u >= 0.0, 0.5, -0.5)
    r = (u + offs).astype(jnp.int32).astype(jnp.float32)
    a = (u - r) * 2.0
    a = jnp.where(a > 0.5, 1.0 - a, jnp.where(a < -0.5, -1.0 - a, a))
    z = a * PI
    z2 = z * z
    p = ((-1.9841270e-4 * z2 + 8.3333338e-3) * z2 + (-1.6666667e-1)) * z2 + 1.0
    return z * p


def _encoder_body(x0_hbm, x1_hbm, x2_hbm, tab0_hbm, tab1_hbm, bnd_hbm, out_hbm,
                  xv, xnv, idxb, wb, rows0, rows1, outb, bvm, sh0, sh1,
                  sem0, sem1, semx):
    wid = lax.axis_index("s") * NUM_CORES + lax.axis_index("c")
    xd_hbm = (x0_hbm, x1_hbm, x2_hbm)
    sems = (sem0, sem1)

    pltpu.sync_copy(bnd_hbm, bvm)

    @pl.when(lax.axis_index("s") == 0)
    def _stage():
        for l in range(N_CACHED):
            pltpu.sync_copy(tab0_hbm.at[pl.ds(l * T_ROWS, SH_ROWS[l])],
                            sh0.at[pl.ds(SH_BASE[l], SH_ROWS[l])])
            pltpu.sync_copy(tab1_hbm.at[pl.ds(l * T_ROWS, SH_ROWS[l])],
                            sh1.at[pl.ds(SH_BASE[l], SH_ROWS[l])])

    plsc.subcore_barrier()

    def compute_fire(l, resf, res1, base_row, hashed, p, src0, src1):
        ib = idxb.at[p]
        wbp = wb.at[p]
        for j in range(NGRP):
            s = LANES * j
            xs = [xnv[d][pl.ds(s, LANES)] for d in range(3)]
            pos = [xc * resf for xc in xs]
            p0i = [q.astype(jnp.int32) for q in pos]
            p0f = [q.astype(jnp.float32) for q in p0i]
            fr = [q - r for q, r in zip(pos, p0f)]
            om = [1.0 - f for f in fr]
            if hashed:
                a0, a1, a2 = p0i[0], p0i[1] * P1, p0i[2] * P2
                c0, c1, c2 = a0 + 1, a1 + P1, a2 + P2
            else:
                r1sq = res1 * res1
                a0, a1, a2 = p0i[0] * r1sq, p0i[1] * res1, p0i[2]
                c0, c1, c2 = a0 + r1sq, a1 + res1, a2 + 1
            for corner in range(8):
                bx, by, bz = corner & 1, (corner >> 1) & 1, (corner >> 2) & 1
                tx = c0 if bx else a0
                ty = c1 if by else a1
                tz = c2 if bz else a2
                h = (tx ^ ty ^ tz) if hashed else (tx + ty + tz)
                ib[pl.ds(corner * CHUNK + s, LANES)] = (h & ROW_MASK) + base_row
                w = (fr[0] if bx else om[0]) * (fr[1] if by else om[1])
                w = w * (fr[2] if bz else om[2])
                wbp[pl.ds(corner * CHUNK + s, LANES)] = w
        for corner in range(8):
            idxref = ib.at[pl.ds(corner * CHUNK, CHUNK)]
            pltpu.async_copy(
                src0.at[idxref],
                rows0.at[p].at[pl.ds(corner * CHUNK, CHUNK)], sems[p])
            pltpu.async_copy(
                src1.at[idxref],
                rows1.at[p].at[pl.ds(corner * CHUNK, CHUNK)], sems[p])

    def drain(q):
        pltpu.make_async_copy(
            tab0_hbm.at[pl.ds(0, CB)], rows0.at[q], sems[q]).wait()
        pltpu.make_async_copy(
            tab1_hbm.at[pl.ds(0, CB)], rows1.at[q], sems[q]).wait()

    def accumulate(l, q):
        r0 = rows0.at[q]
        r1 = rows1.at[q]
        wbq = wb.at[q]
        col0 = 12 + 2 * l
        col1 = 13 + 2 * l
        for j in range(NGRP):
            s = LANES * j
            acc0 = None
            acc1 = None
            for corner in range(8):
                off = corner * CHUNK + s
                g0 = r0[pl.ds(off, LANES)]
                g1 = r1[pl.ds(off, LANES)]
                w = wbq[pl.ds(off, LANES)]
                if corner == 0:
                    acc0, acc1 = w * g0, w * g1
                else:
                    acc0, acc1 = acc0 + w * g0, acc1 + w * g1
            outb[col0, pl.ds(s, LANES)] = acc0
            outb[col1, pl.ds(s, LANES)] = acc1

    def chunk_body(i, carry):
        base = wid * PW + i * CHUNK
        cps = [
            pltpu.async_copy(xd_hbm[d].at[pl.ds(base, CHUNK)], xv[d], semx)
            for d in range(3)
        ]
        for cp in cps:
            cp.wait()
        b = bvm[pl.ds(0, LANES)]
        b2 = b + b
        for j in range(NGRP):
            s = LANES * j
            for d in range(3):
                xd = xv[d][pl.ds(s, LANES)]
                xn = jnp.minimum(jnp.maximum((xd + b) / b2, 0.0), 1.0)
                xnv[d][pl.ds(s, LANES)] = xn

        def freq_group(j, c):
            s = LANES * j
            for d in range(3):
                xd = xv[d][pl.ds(s, LANES)]
                for f in range(2):
                    u = xd * 0.5 if f == 0 else xd
                    outb[6 * f + d, pl.ds(s, LANES)] = _sin2pi(u)
                    outb[6 * f + 3 + d, pl.ds(s, LANES)] = _sin2pi(u + 0.25)
            return c

        lax.fori_loop(0, NGRP, freq_group, 0)

        for l in range(_DENSE_LEVELS):
            res = 16 << l
            compute_fire(l, float(res), res + 1, SH_BASE[l], False, l & 1,
                         sh0, sh1)
            if l > 0:
                drain((l - 1) & 1)
                accumulate(l - 1, (l - 1) & 1)
        compute_fire(3, 128.0, None, SH_BASE[3], True, 1, sh0, sh1)
        drain(0)
        accumulate(2, 0)

        def level_pair(li, c):
            l = 4 + 2 * li
            res = jnp.int32(16) << l
            compute_fire(l, res.astype(jnp.float32), None, l * T_ROWS, True, 0,
                         tab0_hbm, tab1_hbm)
            drain(1)
            accumulate(l - 1, 1)
            resn = res + res
            compute_fire(l + 1, resn.astype(jnp.float32), None,
                         (l + 1) * T_ROWS, True, 1, tab0_hbm, tab1_hbm)
            drain(0)
            accumulate(l, 0)
            return c

        lax.fori_loop(0, (NUM_LEVELS - 4) // 2, level_pair, 0)
        drain(1)
        accumulate(NUM_LEVELS - 1, 1)

        pltpu.sync_copy(outb, out_hbm.at[:, pl.ds(base, CHUNK)])
        return carry

    lax.fori_loop(0, NCHUNK, chunk_body, 0)


@functools.partial(
    pl.kernel,
    out_type=jax.ShapeDtypeStruct((OUT_COLS, N_PTS), jnp.float32),
    mesh=plsc.VectorSubcoreMesh(core_axis_name="c", subcore_axis_name="s"),
    compiler_params=pltpu.CompilerParams(use_tc_tiling_on_sc=False),
    scratch_types=[
        [pltpu.VMEM((CHUNK,), jnp.float32)] * 3,
        [pltpu.VMEM((CHUNK,), jnp.float32)] * 3,
        pltpu.VMEM((2, CB), jnp.int32),
        pltpu.VMEM((2, CB), jnp.float32),
        pltpu.VMEM((2, CB), jnp.float32),
        pltpu.VMEM((2, CB), jnp.float32),
        pltpu.VMEM((OUT_COLS, CHUNK), jnp.float32),
        pltpu.VMEM((LANES,), jnp.float32),
        pltpu.VMEM_SHARED((SH_TOTAL,), jnp.float32),
        pltpu.VMEM_SHARED((SH_TOTAL,), jnp.float32),
        pltpu.SemaphoreType.DMA,
        pltpu.SemaphoreType.DMA,
        pltpu.SemaphoreType.DMA,
    ],
)
def _encoder(x0_hbm, x1_hbm, x2_hbm, tab0_hbm, tab1_hbm, bnd_hbm, out_hbm,
             xv, xnv, idxb, wb, rows0, rows1, outb, bvm, sh0, sh1,
             sem0, sem1, semx):
    _encoder_body(x0_hbm, x1_hbm, x2_hbm, tab0_hbm, tab1_hbm, bnd_hbm, out_hbm,
                  xv, xnv, idxb, wb, rows0, rows1, outb, bvm, sh0, sh1,
                  sem0, sem1, semx)


_TB = 2048


def _transpose_block(src_ref, dst_ref):
    dst_ref[...] = jnp.transpose(src_ref[...], (1, 0))


_transpose = pl.pallas_call(
    _transpose_block,
    out_shape=jax.ShapeDtypeStruct((N_PTS, OUT_COLS), jnp.float32),
    grid=(N_PTS // _TB,),
    in_specs=[pl.BlockSpec((OUT_COLS, _TB), lambda i: (0, i))],
    out_specs=pl.BlockSpec((_TB, OUT_COLS), lambda i: (i, 0)),
)


def kernel(x, table, bound):
    xt = jnp.transpose(x)
    tt = jnp.transpose(table)
    bvec = jnp.full((LANES,), bound, dtype=jnp.float32)
    out_t = _encoder(xt[0], xt[1], xt[2], tt[0], tt[1], bvec)
    return _transpose(out_t)


# speedup vs baseline: 1.0013x; 1.0013x over previous
import functools

import numpy as np
import jax
import jax.numpy as jnp
from jax import lax
from jax.experimental import pallas as pl
from jax.experimental.pallas import tpu as pltpu
from jax.experimental.pallas import tpu_sc as plsc

NUM_LEVELS = 16
T_ROWS = 2 ** 19
ROW_MASK = T_ROWS - 1
N_PTS = 1048576
OUT_COLS = 12 + 2 * NUM_LEVELS

P1 = np.int32(np.uint32(2654435761))
P2 = np.int32(805459861)
PI = 3.14159265358979

NUM_CORES = 2
NUM_SUBCORES = 16
NW = NUM_CORES * NUM_SUBCORES
PW = N_PTS // NW
LANES = 16
CHUNK = 128
NGRP = CHUNK // LANES
NCHUNK = PW // CHUNK
CB = 8 * CHUNK

_DENSE_LEVELS = 3

SH_ROWS = (5224, 37064, 278920, T_ROWS)
SH_BASE = (0, 5224, 42288, 321208)
SH_TOTAL = 845496
N_CACHED = 4


def _sin2pi(u):
    offs = jnp.where(u >= 0.0, 0.5, -0.5)
    r = (u + offs).astype(jnp.int32).astype(jnp.float32)
    a = (u - r) * 2.0
    a = jnp.where(a > 0.5, 1.0 - a, jnp.where(a < -0.5, -1.0 - a, a))
    z = a * PI
    z2 = z * z
    p = ((-1.9841270e-4 * z2 + 8.3333338e-3) * z2 + (-1.6666667e-1)) * z2 + 1.0
    return z * p


def _encoder_body(x0_hbm, x1_hbm, x2_hbm, tab0_hbm, tab1_hbm, bnd_hbm, out_hbm,
                  xv, xnv, idxb, wb, rows0, rows1, outb, bvm, sh0, sh1,
                  sem0, sem1, semx):
    wid = lax.axis_index("s") * NUM_CORES + lax.axis_index("c")
    xd_hbm = (x0_hbm, x1_hbm, x2_hbm)
    sems = (sem0, sem1)

    pltpu.sync_copy(bnd_hbm, bvm)

    @pl.when(lax.axis_index("s") == 0)
    def _stage():
        for l in range(N_CACHED):
            pltpu.sync_copy(tab0_hbm.at[pl.ds(l * T_ROWS, SH_ROWS[l])],
                            sh0.at[pl.ds(SH_BASE[l], SH_ROWS[l])])
            pltpu.sync_copy(tab1_hbm.at[pl.ds(l * T_ROWS, SH_ROWS[l])],
                            sh1.at[pl.ds(SH_BASE[l], SH_ROWS[l])])

    plsc.subcore_barrier()

    def compute_fire(l, resf, res1, base_row, hashed, p, src0, src1):
        ib = idxb.at[p]
        wbp = wb.at[p]
        for j in range(NGRP):
            s = LANES * j
            xs = [xnv[d][pl.ds(s, LANES)] for d in range(3)]
            pos = [xc * resf for xc in xs]
            p0i = [q.astype(jnp.int32) for q in pos]
            p0f = [q.astype(jnp.float32) for q in p0i]
            fr = [q - r for q, r in zip(pos, p0f)]
            om = [1.0 - f for f in fr]
            if hashed:
                a0, a1, a2 = p0i[0], p0i[1] * P1, p0i[2] * P2
                c0, c1, c2 = a0 + 1, a1 + P1, a2 + P2
            else:
                r1sq = res1 * res1
                a0, a1, a2 = p0i[0] * r1sq, p0i[1] * res1, p0i[2]
                c0, c1, c2 = a0 + r1sq, a1 + res1, a2 + 1
            for corner in range(8):
                bx, by, bz = corner & 1, (corner >> 1) & 1, (corner >> 2) & 1
                tx = c0 if bx else a0
                ty = c1 if by else a1
                tz = c2 if bz else a2
                h = (tx ^ ty ^ tz) if hashed else (tx + ty + tz)
                ib[pl.ds(corner * CHUNK + s, LANES)] = (h & ROW_MASK) + base_row
                w = (fr[0] if bx else om[0]) * (fr[1] if by else om[1])
                w = w * (fr[2] if bz else om[2])
                wbp[pl.ds(corner * CHUNK + s, LANES)] = w
        for corner in range(8):
            idxref = ib.at[pl.ds(corner * CHUNK, CHUNK)]
            pltpu.async_copy(
                src0.at[idxref],
                rows0.at[p].at[pl.ds(corner * CHUNK, CHUNK)], sems[p])
            pltpu.async_copy(
                src1.at[idxref],
                rows1.at[p].at[pl.ds(corner * CHUNK, CHUNK)], sems[p])

    def drain(q):
        pltpu.make_async_copy(
            tab0_hbm.at[pl.ds(0, CB)], rows0.at[q], sems[q]).wait()
        pltpu.make_async_copy(
            tab1_hbm.at[pl.ds(0, CB)], rows1.at[q], sems[q]).wait()

    def accumulate(l, q):
        r0 = rows0.at[q]
        r1 = rows1.at[q]
        wbq = wb.at[q]
        col0 = 12 + 2 * l
        col1 = 13 + 2 * l
        for j in range(NGRP):
            s = LANES * j
            acc0 = None
            acc1 = None
            for corner in range(8):
                off = corner * CHUNK + s
                g0 = r0[pl.ds(off, LANES)]
                g1 = r1[pl.ds(off, LANES)]
                w = wbq[pl.ds(off, LANES)]
                if corner == 0:
                    acc0, acc1 = w * g0, w * g1
                else:
                    acc0, acc1 = acc0 + w * g0, acc1 + w * g1
            outb[col0, pl.ds(s, LANES)] = acc0
            outb[col1, pl.ds(s, LANES)] = acc1

    def chunk_body(i, carry):
        base = wid * PW + i * CHUNK
        cps = [
            pltpu.async_copy(xd_hbm[d].at[pl.ds(base, CHUNK)], xv[d], semx)
            for d in range(3)
        ]
        for cp in cps:
            cp.wait()
        b = bvm[pl.ds(0, LANES)]
        b2 = b + b
        for j in range(NGRP):
            s = LANES * j
            for d in range(3):
                xd = xv[d][pl.ds(s, LANES)]
                xn = jnp.minimum(jnp.maximum((xd + b) / b2, 0.0), 1.0)
                xnv[d][pl.ds(s, LANES)] = xn

        def freq_group(j, c):
            s = LANES * j
            for d in range(3):
                xd = xv[d][pl.ds(s, LANES)]
                for f in range(2):
                    u = xd * 0.5 if f == 0 else xd
                    outb[6 * f + d, pl.ds(s, LANES)] = _sin2pi(u)
                    outb[6 * f + 3 + d, pl.ds(s, LANES)] = _sin2pi(u + 0.25)
            return c

        lax.fori_loop(0, NGRP, freq_group, 0)

        for l in range(_DENSE_LEVELS):
            res = 16 << l
            compute_fire(l, float(res), res + 1, SH_BASE[l], False, l & 1,
                         sh0, sh1)
            if l > 0:
                drain((l - 1) & 1)
                accumulate(l - 1, (l - 1) & 1)
        compute_fire(3, 128.0, None, SH_BASE[3], True, 1, sh0, sh1)
        drain(0)
        accumulate(2, 0)

        def level_pair(li, c):
            l = 4 + 2 * li
            res = jnp.int32(16) << l
            compute_fire(l, res.astype(jnp.float32), None, l * T_ROWS, True, 0,
                         tab0_hbm, tab1_hbm)
            drain(1)
            accumulate(l - 1, 1)
            resn = res + res
            compute_fire(l + 1, resn.astype(jnp.float32), None,
                         (l + 1) * T_ROWS, True, 1, tab0_hbm, tab1_hbm)
            drain(0)
            accumulate(l, 0)
            return c

        lax.fori_loop(0, (NUM_LEVELS - 4) // 2, level_pair, 0)
        drain(1)
        accumulate(NUM_LEVELS - 1, 1)

        pltpu.sync_copy(outb, out_hbm.at[:, pl.ds(base, CHUNK)])
        return carry

    lax.fori_loop(0, NCHUNK, chunk_body, 0)


@functools.partial(
    pl.kernel,
    out_type=jax.ShapeDtypeStruct((OUT_COLS, N_PTS), jnp.float32),
    mesh=plsc.VectorSubcoreMesh(core_axis_name="c", subcore_axis_name="s"),
    compiler_params=pltpu.CompilerParams(use_tc_tiling_on_sc=False),
    scratch_types=[
        [pltpu.VMEM((CHUNK,), jnp.float32)] * 3,
        [pltpu.VMEM((CHUNK,), jnp.float32)] * 3,
        pltpu.VMEM((2, CB), jnp.int32),
        pltpu.VMEM((2, CB), jnp.float32),
        pltpu.VMEM((2, CB), jnp.float32),
        pltpu.VMEM((2, CB), jnp.float32),
        pltpu.VMEM((OUT_COLS, CHUNK), jnp.float32),
        pltpu.VMEM((LANES,), jnp.float32),
        pltpu.VMEM_SHARED((SH_TOTAL,), jnp.float32),
        pltpu.VMEM_SHARED((SH_TOTAL,), jnp.float32),
        pltpu.SemaphoreType.DMA,
        pltpu.SemaphoreType.DMA,
        pltpu.SemaphoreType.DMA,
    ],
)
def _encoder(x0_hbm, x1_hbm, x2_hbm, tab0_hbm, tab1_hbm, bnd_hbm, out_hbm,
             xv, xnv, idxb, wb, rows0, rows1, outb, bvm, sh0, sh1,
             sem0, sem1, semx):
    _encoder_body(x0_hbm, x1_hbm, x2_hbm, tab0_hbm, tab1_hbm, bnd_hbm, out_hbm,
                  xv, xnv, idxb, wb, rows0, rows1, outb, bvm, sh0, sh1,
                  sem0, sem1, semx)


_TB = 8192


def _transpose_block(src_ref, eye_ref, dst_ref):
    dst_ref[...] = jax.lax.dot_general(
        src_ref[...], eye_ref[...], (((0,), (0,)), ((), ())),
        precision=lax.Precision.HIGHEST,
        preferred_element_type=jnp.float32)


_transpose_call = pl.pallas_call(
    _transpose_block,
    out_shape=jax.ShapeDtypeStruct((N_PTS, OUT_COLS), jnp.float32),
    grid=(N_PTS // _TB,),
    in_specs=[pl.BlockSpec((OUT_COLS, _TB), lambda i: (0, i)),
              pl.BlockSpec((OUT_COLS, OUT_COLS), lambda i: (0, 0))],
    out_specs=pl.BlockSpec((_TB, OUT_COLS), lambda i: (i, 0)),
)


def _transpose(out_t):
    return _transpose_call(out_t, jnp.eye(OUT_COLS, dtype=jnp.float32))


def kernel(x, table, bound):
    xt = jnp.transpose(x)
    tt = jnp.transpose(table)
    bvec = jnp.full((LANES,), bound, dtype=jnp.float32)
    out_t = _encoder(xt[0], xt[1], xt[2], tt[0], tt[1], bvec)
    return _transpose(out_t)


# speedup vs baseline: 1.0013x; 1.0000x over previous
import functools

import numpy as np
import jax
import jax.numpy as jnp
from jax import lax
from jax.experimental import pallas as pl
from jax.experimental.pallas import tpu as pltpu
from jax.experimental.pallas import tpu_sc as plsc

NUM_LEVELS = 16
T_ROWS = 2 ** 19
ROW_MASK = T_ROWS - 1
N_PTS = 1048576
OUT_COLS = 12 + 2 * NUM_LEVELS

P1 = np.int32(np.uint32(2654435761))
P2 = np.int32(805459861)
PI = 3.14159265358979

NUM_CORES = 2
NUM_SUBCORES = 16
NW = NUM_CORES * NUM_SUBCORES
PW = N_PTS // NW
LANES = 16
CHUNK = 128
NGRP = CHUNK // LANES
NCHUNK = PW // CHUNK
CB = 8 * CHUNK

_DENSE_LEVELS = 3

SH_ROWS = (5224, 37064, 278920, T_ROWS)
SH_BASE = (0, 5224, 42288, 321208)
SH_TOTAL = 845496
N_CACHED = 4


def _sin2pi(u):
    offs = jnp.where(u >= 0.0, 0.5, -0.5)
    r = (u + offs).astype(jnp.int32).astype(jnp.float32)
    a = (u - r) * 2.0
    a = jnp.where(a > 0.5, 1.0 - a, jnp.where(a < -0.5, -1.0 - a, a))
    z = a * PI
    z2 = z * z
    p = ((-1.9841270e-4 * z2 + 8.3333338e-3) * z2 + (-1.6666667e-1)) * z2 + 1.0
    return z * p


def _encoder_body(x0_hbm, x1_hbm, x2_hbm, tab0_hbm, tab1_hbm, bnd_hbm, out_hbm,
                  xv, xnv, idxb, wb, rows0, rows1, outb, bvm, sh0, sh1,
                  sem0, sem1, semx):
    wid = lax.axis_index("s") * NUM_CORES + lax.axis_index("c")
    xd_hbm = (x0_hbm, x1_hbm, x2_hbm)
    sems = (sem0, sem1)

    pltpu.sync_copy(bnd_hbm, bvm)

    @pl.when(lax.axis_index("s") == 0)
    def _stage():
        for l in range(N_CACHED):
            pltpu.sync_copy(tab0_hbm.at[pl.ds(l * T_ROWS, SH_ROWS[l])],
                            sh0.at[pl.ds(SH_BASE[l], SH_ROWS[l])])
            pltpu.sync_copy(tab1_hbm.at[pl.ds(l * T_ROWS, SH_ROWS[l])],
                            sh1.at[pl.ds(SH_BASE[l], SH_ROWS[l])])

    plsc.subcore_barrier()

    def compute_fire(l, resf, res1, base_row, hashed, p, src0, src1):
        ib = idxb.at[p]
        wbp = wb.at[p]
        for j in range(NGRP):
            s = LANES * j
            xs = [xnv[d][pl.ds(s, LANES)] for d in range(3)]
            pos = [xc * resf for xc in xs]
            p0i = [q.astype(jnp.int32) for q in pos]
            p0f = [q.astype(jnp.float32) for q in p0i]
            fr = [q - r for q, r in zip(pos, p0f)]
            om = [1.0 - f for f in fr]
            if hashed:
                a0, a1, a2 = p0i[0], p0i[1] * P1, p0i[2] * P2
                c0, c1, c2 = a0 + 1, a1 + P1, a2 + P2
            else:
                r1sq = res1 * res1
                a0, a1, a2 = p0i[0] * r1sq, p0i[1] * res1, p0i[2]
                c0, c1, c2 = a0 + r1sq, a1 + res1, a2 + 1
            for corner in range(8):
                bx, by, bz = corner & 1, (corner >> 1) & 1, (corner >> 2) & 1
                tx = c0 if bx else a0
                ty = c1 if by else a1
                tz = c2 if bz else a2
                h = (tx ^ ty ^ tz) if hashed else (tx + ty + tz)
                ib[pl.ds(corner * CHUNK + s, LANES)] = (h & ROW_MASK) + base_row
                w = (fr[0] if bx else om[0]) * (fr[1] if by else om[1])
                w = w * (fr[2] if bz else om[2])
                wbp[pl.ds(corner * CHUNK + s, LANES)] = w
        for corner in range(8):
            idxref = ib.at[pl.ds(corner * CHUNK, CHUNK)]
            pltpu.async_copy(
                src0.at[idxref],
                rows0.at[p].at[pl.ds(corner * CHUNK, CHUNK)], sems[p])
            pltpu.async_copy(
                src1.at[idxref],
                rows1.at[p].at[pl.ds(corner * CHUNK, CHUNK)], sems[p])

    def drain(q):
        pltpu.make_async_copy(
            tab0_hbm.at[pl.ds(0, CB)], rows0.at[q], sems[q]).wait()
        pltpu.make_async_copy(
            tab1_hbm.at[pl.ds(0, CB)], rows1.at[q], sems[q]).wait()

    def accumulate(l, q):
        r0 = rows0.at[q]
        r1 = rows1.at[q]
        wbq = wb.at[q]
        col0 = 12 + 2 * l
        col1 = 13 + 2 * l
        for j in range(NGRP):
            s = LANES * j
            acc0 = None
            acc1 = None
            for corner in range(8):
                off = corner * CHUNK + s
                g0 = r0[pl.ds(off, LANES)]
                g1 = r1[pl.ds(off, LANES)]
                w = wbq[pl.ds(off, LANES)]
                if corner == 0:
                    acc0, acc1 = w * g0, w * g1
                else:
                    acc0, acc1 = acc0 + w * g0, acc1 + w * g1
            outb[col0, pl.ds(s, LANES)] = acc0
            outb[col1, pl.ds(s, LANES)] = acc1

    def chunk_body(i, carry):
        base = wid * PW + i * CHUNK
        cps = [
            pltpu.async_copy(xd_hbm[d].at[pl.ds(base, CHUNK)], xv[d], semx)
            for d in range(3)
        ]
        for cp in cps:
            cp.wait()
        b = bvm[pl.ds(0, LANES)]
        b2 = b + b
        for j in range(NGRP):
            s = LANES * j
            for d in range(3):
                xd = xv[d][pl.ds(s, LANES)]
                xn = jnp.minimum(jnp.maximum((xd + b) / b2, 0.0), 1.0)
                xnv[d][pl.ds(s, LANES)] = xn

        def freq_group(j, c):
            s = LANES * j
            for d in range(3):
                xd = xv[d][pl.ds(s, LANES)]
                for f in range(2):
                    u = xd * 0.5 if f == 0 else xd
                    outb[6 * f + d, pl.ds(s, LANES)] = _sin2pi(u)
                    outb[6 * f + 3 + d, pl.ds(s, LANES)] = _sin2pi(u + 0.25)
            return c

        lax.fori_loop(0, NGRP, freq_group, 0)

        for l in range(_DENSE_LEVELS):
            res = 16 << l
            compute_fire(l, float(res), res + 1, SH_BASE[l], False, l & 1,
                         sh0, sh1)
            if l > 0:
                drain((l - 1) & 1)
                accumulate(l - 1, (l - 1) & 1)
        compute_fire(3, 128.0, None, SH_BASE[3], True, 1, sh0, sh1)
        drain(0)
        accumulate(2, 0)

        def level_pair(li, c):
            l = 4 + 2 * li
            res = jnp.int32(16) << l
            compute_fire(l, res.astype(jnp.float32), None, l * T_ROWS, True, 0,
                         tab0_hbm, tab1_hbm)
            drain(1)
            accumulate(l - 1, 1)
            resn = res + res
            compute_fire(l + 1, resn.astype(jnp.float32), None,
                         (l + 1) * T_ROWS, True, 1, tab0_hbm, tab1_hbm)
            drain(0)
            accumulate(l, 0)
            return c

        lax.fori_loop(0, (NUM_LEVELS - 4) // 2, level_pair, 0)
        drain(1)
        accumulate(NUM_LEVELS - 1, 1)

        ocps = [
            pltpu.async_copy(
                outb.at[col],
                out_hbm.at[pl.ds(col * N_PTS + base, CHUNK)], semx)
            for col in range(OUT_COLS)
        ]
        for cp in ocps:
            cp.wait()
        return carry

    lax.fori_loop(0, NCHUNK, chunk_body, 0)


@functools.partial(
    pl.kernel,
    out_type=jax.ShapeDtypeStruct((OUT_COLS * N_PTS,), jnp.float32),
    mesh=plsc.VectorSubcoreMesh(core_axis_name="c", subcore_axis_name="s"),
    compiler_params=pltpu.CompilerParams(use_tc_tiling_on_sc=False),
    scratch_types=[
        [pltpu.VMEM((CHUNK,), jnp.float32)] * 3,
        [pltpu.VMEM((CHUNK,), jnp.float32)] * 3,
        pltpu.VMEM((2, CB), jnp.int32),
        pltpu.VMEM((2, CB), jnp.float32),
        pltpu.VMEM((2, CB), jnp.float32),
        pltpu.VMEM((2, CB), jnp.float32),
        pltpu.VMEM((OUT_COLS, CHUNK), jnp.float32),
        pltpu.VMEM((LANES,), jnp.float32),
        pltpu.VMEM_SHARED((SH_TOTAL,), jnp.float32),
        pltpu.VMEM_SHARED((SH_TOTAL,), jnp.float32),
        pltpu.SemaphoreType.DMA,
        pltpu.SemaphoreType.DMA,
        pltpu.SemaphoreType.DMA,
    ],
)
def _encoder(x0_hbm, x1_hbm, x2_hbm, tab0_hbm, tab1_hbm, bnd_hbm, out_hbm,
             xv, xnv, idxb, wb, rows0, rows1, outb, bvm, sh0, sh1,
             sem0, sem1, semx):
    _encoder_body(x0_hbm, x1_hbm, x2_hbm, tab0_hbm, tab1_hbm, bnd_hbm, out_hbm,
                  xv, xnv, idxb, wb, rows0, rows1, outb, bvm, sh0, sh1,
                  sem0, sem1, semx)


_TB = 8192


def _transpose_block(src_ref, eye_ref, dst_ref):
    dst_ref[...] = jax.lax.dot_general(
        src_ref[...], eye_ref[...], (((0,), (0,)), ((), ())),
        precision=lax.Precision.HIGHEST,
        preferred_element_type=jnp.float32)


_transpose_call = pl.pallas_call(
    _transpose_block,
    out_shape=jax.ShapeDtypeStruct((N_PTS, OUT_COLS), jnp.float32),
    grid=(N_PTS // _TB,),
    in_specs=[pl.BlockSpec((OUT_COLS, _TB), lambda i: (0, i)),
              pl.BlockSpec((OUT_COLS, OUT_COLS), lambda i: (0, 0))],
    out_specs=pl.BlockSpec((_TB, OUT_COLS), lambda i: (i, 0)),
)


def _transpose(out_t):
    return _transpose_call(out_t, jnp.eye(OUT_COLS, dtype=jnp.float32))


def kernel(x, table, bound):
    xt = jnp.transpose(x)
    tt = jnp.transpose(table)
    bvec = jnp.full((LANES,), bound, dtype=jnp.float32)
    out_t = _encoder(xt[0], xt[1], xt[2], tt[0], tt[1], bvec)
    return _transpose(jnp.reshape(out_t, (OUT_COLS, N_PTS)))


# speedup vs baseline: 1.0567x; 1.0553x over previous
import functools

import numpy as np
import jax
import jax.numpy as jnp
from jax import lax
from jax.experimental import pallas as pl
from jax.experimental.pallas import tpu as pltpu
from jax.experimental.pallas import tpu_sc as plsc

NUM_LEVELS = 16
T_ROWS = 2 ** 19
ROW_MASK = T_ROWS - 1
N_PTS = 1048576
OUT_COLS = 12 + 2 * NUM_LEVELS

P1 = np.int32(np.uint32(2654435761))
P2 = np.int32(805459861)
PI = 3.14159265358979

NUM_CORES = 2
NUM_SUBCORES = 16
NW = NUM_CORES * NUM_SUBCORES
PW = N_PTS // NW
LANES = 16
CHUNK = 128
NGRP = CHUNK // LANES
NCHUNK = PW // CHUNK
CB = 8 * CHUNK

_DENSE_LEVELS = 3

SH_ROWS = (5224, 37064, 278920, T_ROWS)
SH_BASE = (0, 5224, 42288, 321208)
SH_TOTAL = 845496
N_CACHED = 4


def _sin2pi(u):
    offs = jnp.where(u >= 0.0, 0.5, -0.5)
    r = (u + offs).astype(jnp.int32).astype(jnp.float32)
    a = (u - r) * 2.0
    a = jnp.where(a > 0.5, 1.0 - a, jnp.where(a < -0.5, -1.0 - a, a))
    z = a * PI
    z2 = z * z
    p = ((-1.9841270e-4 * z2 + 8.3333338e-3) * z2 + (-1.6666667e-1)) * z2 + 1.0
    return z * p


def _encoder_body(x0_hbm, x1_hbm, x2_hbm, tab0_hbm, tab1_hbm, bnd_hbm, out_hbm,
                  xv, xnv, idxb, wb, rows0, rows1, outb, bvm, sh0, sh1,
                  sem0, sem1, semx):
    wid = lax.axis_index("s") * NUM_CORES + lax.axis_index("c")
    xd_hbm = (x0_hbm, x1_hbm, x2_hbm)
    sems = (sem0, sem1)

    pltpu.sync_copy(bnd_hbm, bvm)

    @pl.when(lax.axis_index("s") == 0)
    def _stage():
        for l in range(N_CACHED):
            pltpu.sync_copy(tab0_hbm.at[pl.ds(l * T_ROWS, SH_ROWS[l])],
                            sh0.at[pl.ds(SH_BASE[l], SH_ROWS[l])])
            pltpu.sync_copy(tab1_hbm.at[pl.ds(l * T_ROWS, SH_ROWS[l])],
                            sh1.at[pl.ds(SH_BASE[l], SH_ROWS[l])])

    plsc.subcore_barrier()

    def compute_fire(l, resf, res1, base_row, hashed, p, src0, src1):
        ib = idxb.at[p]
        wbp = wb.at[p]
        for j in range(NGRP):
            s = LANES * j
            xs = [xnv[d][pl.ds(s, LANES)] for d in range(3)]
            pos = [xc * resf for xc in xs]
            p0i = [q.astype(jnp.int32) for q in pos]
            p0f = [q.astype(jnp.float32) for q in p0i]
            fr = [q - r for q, r in zip(pos, p0f)]
            om = [1.0 - f for f in fr]
            if hashed:
                a0, a1, a2 = p0i[0], p0i[1] * P1, p0i[2] * P2
                c0, c1, c2 = a0 + 1, a1 + P1, a2 + P2
            else:
                r1sq = res1 * res1
                a0, a1, a2 = p0i[0] * r1sq, p0i[1] * res1, p0i[2]
                c0, c1, c2 = a0 + r1sq, a1 + res1, a2 + 1
            for corner in range(8):
                bx, by, bz = corner & 1, (corner >> 1) & 1, (corner >> 2) & 1
                tx = c0 if bx else a0
                ty = c1 if by else a1
                tz = c2 if bz else a2
                h = (tx ^ ty ^ tz) if hashed else (tx + ty + tz)
                ib[pl.ds(corner * CHUNK + s, LANES)] = (h & ROW_MASK) + base_row
                w = (fr[0] if bx else om[0]) * (fr[1] if by else om[1])
                w = w * (fr[2] if bz else om[2])
                wbp[pl.ds(corner * CHUNK + s, LANES)] = w
        for corner in range(8):
            idxref = ib.at[pl.ds(corner * CHUNK, CHUNK)]
            pltpu.async_copy(
                src0.at[idxref],
                rows0.at[p].at[pl.ds(corner * CHUNK, CHUNK)], sems[p])
            pltpu.async_copy(
                src1.at[idxref],
                rows1.at[p].at[pl.ds(corner * CHUNK, CHUNK)], sems[p])

    def drain(q):
        pltpu.make_async_copy(
            tab0_hbm.at[pl.ds(0, CB)], rows0.at[q], sems[q]).wait()
        pltpu.make_async_copy(
            tab1_hbm.at[pl.ds(0, CB)], rows1.at[q], sems[q]).wait()

    def accumulate(l, q):
        r0 = rows0.at[q]
        r1 = rows1.at[q]
        wbq = wb.at[q]
        col0 = 12 + 2 * l
        col1 = 13 + 2 * l
        for j in range(NGRP):
            s = LANES * j
            acc0 = None
            acc1 = None
            for corner in range(8):
                off = corner * CHUNK + s
                g0 = r0[pl.ds(off, LANES)]
                g1 = r1[pl.ds(off, LANES)]
                w = wbq[pl.ds(off, LANES)]
                if corner == 0:
                    acc0, acc1 = w * g0, w * g1
                else:
                    acc0, acc1 = acc0 + w * g0, acc1 + w * g1
            outb[col0, pl.ds(s, LANES)] = acc0
            outb[col1, pl.ds(s, LANES)] = acc1

    def chunk_body(i, carry):
        base = wid * PW + i * CHUNK
        cps = [
            pltpu.async_copy(xd_hbm[d].at[pl.ds(base, CHUNK)], xv[d], semx)
            for d in range(3)
        ]
        for cp in cps:
            cp.wait()
        b = bvm[pl.ds(0, LANES)]
        b2 = b + b
        for j in range(NGRP):
            s = LANES * j
            for d in range(3):
                xd = xv[d][pl.ds(s, LANES)]
                xn = jnp.minimum(jnp.maximum((xd + b) / b2, 0.0), 1.0)
                xnv[d][pl.ds(s, LANES)] = xn

        def freq_group(j, c):
            s = LANES * j
            for d in range(3):
                xd = xv[d][pl.ds(s, LANES)]
                for f in range(2):
                    u = xd * 0.5 if f == 0 else xd
                    outb[6 * f + d, pl.ds(s, LANES)] = _sin2pi(u)
                    outb[6 * f + 3 + d, pl.ds(s, LANES)] = _sin2pi(u + 0.25)
            return c

        lax.fori_loop(0, NGRP, freq_group, 0)

        for l in range(_DENSE_LEVELS):
            res = 16 << l
            compute_fire(l, float(res), res + 1, SH_BASE[l], False, l & 1,
                         sh0, sh1)
            if l > 0:
                drain((l - 1) & 1)
                accumulate(l - 1, (l - 1) & 1)
        compute_fire(3, 128.0, None, SH_BASE[3], True, 1, sh0, sh1)
        drain(0)
        accumulate(2, 0)

        def level_pair(li, c):
            l = 4 + 2 * li
            res = jnp.int32(16) << l
            compute_fire(l, res.astype(jnp.float32), None, l * T_ROWS, True, 0,
                         tab0_hbm, tab1_hbm)
            drain(1)
            accumulate(l - 1, 1)
            resn = res + res
            compute_fire(l + 1, resn.astype(jnp.float32), None,
                         (l + 1) * T_ROWS, True, 1, tab0_hbm, tab1_hbm)
            drain(0)
            accumulate(l, 0)
            return c

        lax.fori_loop(0, (NUM_LEVELS - 4) // 2, level_pair, 0)
        drain(1)
        accumulate(NUM_LEVELS - 1, 1)

        pltpu.sync_copy(outb, out_hbm.at[:, pl.ds(base, CHUNK)])
        return carry

    lax.fori_loop(0, NCHUNK, chunk_body, 0)


@functools.partial(
    pl.kernel,
    out_type=jax.ShapeDtypeStruct((OUT_COLS, N_PTS), jnp.float32),
    mesh=plsc.VectorSubcoreMesh(core_axis_name="c", subcore_axis_name="s"),
    compiler_params=pltpu.CompilerParams(use_tc_tiling_on_sc=False),
    scratch_types=[
        [pltpu.VMEM((CHUNK,), jnp.float32)] * 3,
        [pltpu.VMEM((CHUNK,), jnp.float32)] * 3,
        pltpu.VMEM((2, CB), jnp.int32),
        pltpu.VMEM((2, CB), jnp.float32),
        pltpu.VMEM((2, CB), jnp.float32),
        pltpu.VMEM((2, CB), jnp.float32),
        pltpu.VMEM((OUT_COLS, CHUNK), jnp.float32),
        pltpu.VMEM((LANES,), jnp.float32),
        pltpu.VMEM_SHARED((SH_TOTAL,), jnp.float32),
        pltpu.VMEM_SHARED((SH_TOTAL,), jnp.float32),
        pltpu.SemaphoreType.DMA,
        pltpu.SemaphoreType.DMA,
        pltpu.SemaphoreType.DMA,
    ],
)
def _encoder(x0_hbm, x1_hbm, x2_hbm, tab0_hbm, tab1_hbm, bnd_hbm, out_hbm,
             xv, xnv, idxb, wb, rows0, rows1, outb, bvm, sh0, sh1,
             sem0, sem1, semx):
    _encoder_body(x0_hbm, x1_hbm, x2_hbm, tab0_hbm, tab1_hbm, bnd_hbm, out_hbm,
                  xv, xnv, idxb, wb, rows0, rows1, outb, bvm, sh0, sh1,
                  sem0, sem1, semx)


def kernel(x, table, bound):
    xt = jnp.transpose(x)
    tt = jnp.transpose(table)
    bvec = jnp.full((LANES,), bound, dtype=jnp.float32)
    out_t = _encoder(xt[0], xt[1], xt[2], tt[0], tt[1], bvec)
    return jnp.transpose(out_t)


# speedup vs baseline: 1.4909x; 1.4109x over previous
import functools

import numpy as np
import jax
import jax.numpy as jnp
from jax import lax
from jax.experimental import pallas as pl
from jax.experimental.pallas import tpu as pltpu
from jax.experimental.pallas import tpu_sc as plsc

NUM_LEVELS = 16
T_ROWS = 2 ** 19
ROW_MASK = T_ROWS - 1
N_PTS = 1048576
OUT_COLS = 12 + 2 * NUM_LEVELS

P1 = np.int32(np.uint32(2654435761))
P2 = np.int32(805459861)
PI = 3.14159265358979

NUM_CORES = 2
NUM_SUBCORES = 16
NW = NUM_CORES * NUM_SUBCORES
PW = N_PTS // NW
LANES = 16
CHUNK = 128
NGRP = CHUNK // LANES
NCHUNK = PW // CHUNK
CB = 8 * CHUNK

_DENSE_LEVELS = 3

SH_ROWS = (5224, 37064, 278920, T_ROWS)
SH_BASE = (0, 5224, 42288, 321208)
SH_TOTAL = 845496
N_CACHED = 4


def _sin2pi(u):
    offs = jnp.where(u >= 0.0, 0.5, -0.5)
    r = (u + offs).astype(jnp.int32).astype(jnp.float32)
    a = (u - r) * 2.0
    a = jnp.where(a > 0.5, 1.0 - a, jnp.where(a < -0.5, -1.0 - a, a))
    z = a * PI
    z2 = z * z
    p = ((-1.9841270e-4 * z2 + 8.3333338e-3) * z2 + (-1.6666667e-1)) * z2 + 1.0
    return z * p


def _encoder_body(x0_hbm, x1_hbm, x2_hbm, tab0_hbm, tab1_hbm, bnd_hbm, out_hbm,
                  xv, xnv, idxb, wb, rows0, rows1, outb, bvm, sh0, sh1,
                  sem0, sem1, semx):
    wid = lax.axis_index("s") * NUM_CORES + lax.axis_index("c")
    xd_hbm = (x0_hbm, x1_hbm, x2_hbm)
    sems = (sem0, sem1)

    pltpu.sync_copy(bnd_hbm, bvm)

    @pl.when(lax.axis_index("s") == 0)
    def _stage():
        for l in range(N_CACHED):
            pltpu.sync_copy(tab0_hbm.at[pl.ds(l * T_ROWS, SH_ROWS[l])],
                            sh0.at[pl.ds(SH_BASE[l], SH_ROWS[l])])
            pltpu.sync_copy(tab1_hbm.at[pl.ds(l * T_ROWS, SH_ROWS[l])],
                            sh1.at[pl.ds(SH_BASE[l], SH_ROWS[l])])

    plsc.subcore_barrier()

    def compute_fire(l, resf, res1, base_row, hashed, p, src0, src1):
        ib = idxb.at[p]
        wbp = wb.at[p]
        for j in range(NGRP):
            s = LANES * j
            xs = [xnv[d][pl.ds(s, LANES)] for d in range(3)]
            pos = [xc * resf for xc in xs]
            p0i = [q.astype(jnp.int32) for q in pos]
            p0f = [q.astype(jnp.float32) for q in p0i]
            fr = [q - r for q, r in zip(pos, p0f)]
            om = [1.0 - f for f in fr]
            if hashed:
                a0, a1, a2 = p0i[0], p0i[1] * P1, p0i[2] * P2
                c0, c1, c2 = a0 + 1, a1 + P1, a2 + P2
            else:
                r1sq = res1 * res1
                a0, a1, a2 = p0i[0] * r1sq, p0i[1] * res1, p0i[2]
                c0, c1, c2 = a0 + r1sq, a1 + res1, a2 + 1
            for corner in range(8):
                bx, by, bz = corner & 1, (corner >> 1) & 1, (corner >> 2) & 1
                tx = c0 if bx else a0
                ty = c1 if by else a1
                tz = c2 if bz else a2
                h = (tx ^ ty ^ tz) if hashed else (tx + ty + tz)
                ib[pl.ds(corner * CHUNK + s, LANES)] = (h & ROW_MASK) + base_row
                w = (fr[0] if bx else om[0]) * (fr[1] if by else om[1])
                w = w * (fr[2] if bz else om[2])
                wbp[pl.ds(corner * CHUNK + s, LANES)] = w
        for corner in range(8):
            idxref = ib.at[pl.ds(corner * CHUNK, CHUNK)]
            pltpu.async_copy(
                src0.at[idxref],
                rows0.at[p].at[pl.ds(corner * CHUNK, CHUNK)], sems[p])
            pltpu.async_copy(
                src1.at[idxref],
                rows1.at[p].at[pl.ds(corner * CHUNK, CHUNK)], sems[p])

    def drain(q):
        pltpu.make_async_copy(
            tab0_hbm.at[pl.ds(0, CB)], rows0.at[q], sems[q]).wait()
        pltpu.make_async_copy(
            tab1_hbm.at[pl.ds(0, CB)], rows1.at[q], sems[q]).wait()

    def accumulate(l, q):
        r0 = rows0.at[q]
        r1 = rows1.at[q]
        wbq = wb.at[q]
        col0 = 12 + 2 * l
        col1 = 13 + 2 * l
        for j in range(NGRP):
            s = LANES * j
            acc0 = None
            acc1 = None
            for corner in range(8):
                off = corner * CHUNK + s
                g0 = r0[pl.ds(off, LANES)]
                g1 = r1[pl.ds(off, LANES)]
                w = wbq[pl.ds(off, LANES)]
                if corner == 0:
                    acc0, acc1 = w * g0, w * g1
                else:
                    acc0, acc1 = acc0 + w * g0, acc1 + w * g1
            outb[col0, pl.ds(s, LANES)] = acc0
            outb[col1, pl.ds(s, LANES)] = acc1

    def chunk_body(i, carry):
        base = wid * PW + i * CHUNK
        cps = [
            pltpu.async_copy(xd_hbm[d].at[pl.ds(base, CHUNK)], xv[d], semx)
            for d in range(3)
        ]
        for cp in cps:
            cp.wait()
        b = bvm[pl.ds(0, LANES)]
        b2 = b + b
        for j in range(NGRP):
            s = LANES * j
            for d in range(3):
                xd = xv[d][pl.ds(s, LANES)]
                xn = jnp.minimum(jnp.maximum((xd + b) / b2, 0.0), 1.0)
                xnv[d][pl.ds(s, LANES)] = xn

        def freq_group(j, c):
            s = LANES * j
            for d in range(3):
                xd = xv[d][pl.ds(s, LANES)]
                for f in range(2):
                    u = xd * 0.5 if f == 0 else xd
                    outb[6 * f + d, pl.ds(s, LANES)] = _sin2pi(u)
                    outb[6 * f + 3 + d, pl.ds(s, LANES)] = _sin2pi(u + 0.25)
            return c

        lax.fori_loop(0, NGRP, freq_group, 0)

        for l in range(_DENSE_LEVELS):
            res = 16 << l
            compute_fire(l, float(res), res + 1, SH_BASE[l], False, l & 1,
                         sh0, sh1)
            if l > 0:
                drain((l - 1) & 1)
                accumulate(l - 1, (l - 1) & 1)
        compute_fire(3, 128.0, None, SH_BASE[3], True, 1, sh0, sh1)
        drain(0)
        accumulate(2, 0)

        def level_pair(li, c):
            l = 4 + 2 * li
            res = jnp.int32(16) << l
            compute_fire(l, res.astype(jnp.float32), None, l * T_ROWS, True, 0,
                         tab0_hbm, tab1_hbm)
            drain(1)
            accumulate(l - 1, 1)
            resn = res + res
            compute_fire(l + 1, resn.astype(jnp.float32), None,
                         (l + 1) * T_ROWS, True, 1, tab0_hbm, tab1_hbm)
            drain(0)
            accumulate(l, 0)
            return c

        lax.fori_loop(0, (NUM_LEVELS - 4) // 2, level_pair, 0)
        drain(1)
        accumulate(NUM_LEVELS - 1, 1)

        pltpu.sync_copy(outb, out_hbm.at[wid * NCHUNK + i])
        return carry

    lax.fori_loop(0, NCHUNK, chunk_body, 0)


@functools.partial(
    pl.kernel,
    out_type=jax.ShapeDtypeStruct((N_PTS // CHUNK, OUT_COLS, CHUNK),
                                   jnp.float32),
    mesh=plsc.VectorSubcoreMesh(core_axis_name="c", subcore_axis_name="s"),
    compiler_params=pltpu.CompilerParams(use_tc_tiling_on_sc=False),
    scratch_types=[
        [pltpu.VMEM((CHUNK,), jnp.float32)] * 3,
        [pltpu.VMEM((CHUNK,), jnp.float32)] * 3,
        pltpu.VMEM((2, CB), jnp.int32),
        pltpu.VMEM((2, CB), jnp.float32),
        pltpu.VMEM((2, CB), jnp.float32),
        pltpu.VMEM((2, CB), jnp.float32),
        pltpu.VMEM((OUT_COLS, CHUNK), jnp.float32),
        pltpu.VMEM((LANES,), jnp.float32),
        pltpu.VMEM_SHARED((SH_TOTAL,), jnp.float32),
        pltpu.VMEM_SHARED((SH_TOTAL,), jnp.float32),
        pltpu.SemaphoreType.DMA,
        pltpu.SemaphoreType.DMA,
        pltpu.SemaphoreType.DMA,
    ],
)
def _encoder(x0_hbm, x1_hbm, x2_hbm, tab0_hbm, tab1_hbm, bnd_hbm, out_hbm,
             xv, xnv, idxb, wb, rows0, rows1, outb, bvm, sh0, sh1,
             sem0, sem1, semx):
    _encoder_body(x0_hbm, x1_hbm, x2_hbm, tab0_hbm, tab1_hbm, bnd_hbm, out_hbm,
                  xv, xnv, idxb, wb, rows0, rows1, outb, bvm, sh0, sh1,
                  sem0, sem1, semx)


def kernel(x, table, bound):
    xt = jnp.transpose(x)
    tt = jnp.transpose(table)
    bvec = jnp.full((LANES,), bound, dtype=jnp.float32)
    out_t = _encoder(xt[0], xt[1], xt[2], tt[0], tt[1], bvec)
    return jnp.reshape(jnp.transpose(out_t, (0, 2, 1)), (N_PTS, OUT_COLS))
